# baseline (speedup 1.0000x reference)
import functools

import jax
import jax.numpy as jnp
from jax import lax
from jax.experimental import pallas as pl
from jax.experimental.pallas import tpu as pltpu
from jax.experimental.pallas import tpu_sc as plsc

LANES = 16


def _sc_gather_sum(ids, table):
    L, B = ids.shape
    _, D = table.shape
    NC, NS = 2, 16
    NW = NC * NS
    bpw = B // NW
    CV = D // LANES
    JU = 8

    mesh = plsc.VectorSubcoreMesh(core_axis_name="c", subcore_axis_name="s")

    @functools.partial(
        pl.kernel,
        out_type=jax.ShapeDtypeStruct((B, D), jnp.float32),
        mesh=mesh,
        scratch_types=[
            pltpu.VMEM((L, bpw), jnp.int32),
            pltpu.VMEM((2, bpw, D), jnp.float32),
            pltpu.VMEM((bpw, D), jnp.float32),
            pltpu.SemaphoreType.DMA,
            pltpu.SemaphoreType.DMA,
        ],
        compiler_params=pltpu.CompilerParams(use_tc_tiling_on_sc=False),
    )
    def k(ids_hbm, table_hbm, out_hbm, idx_v, rows_v, acc_v, sem0, sem1):
        wid = lax.axis_index("s") * NC + lax.axis_index("c")
        base = wid * bpw
        pltpu.sync_copy(ids_hbm.at[:, pl.ds(base, bpw)], idx_v)
        sems = (sem0, sem1)

        def gather(l, buf):
            pltpu.async_copy(table_hbm.at[idx_v.at[l]], rows_v.at[buf], sems[buf])

        def wait_gather(l, buf):
            pltpu.make_async_copy(
                table_hbm.at[idx_v.at[l]], rows_v.at[buf], sems[buf]
            ).wait()

        def consume(buf, first):
            def jbody(i, _):
                for u in range(JU):
                    j = i * JU + u
                    for c in range(CV):
                        sl = pl.ds(c * LANES, LANES)
                        x = rows_v[buf, j, sl]
                        if first:
                            acc_v[j, sl] = x
                        else:
                            plsc.addupdate(acc_v.at[j, sl], x)
                return 0

            lax.fori_loop(0, bpw // JU, jbody, 0)

        gather(0, 0)
        gather(1, 1)
        wait_gather(0, 0)
        gather(2, 0)
        consume(0, first=True)
        wait_gather(1, 1)
        gather(3, 1)
        consume(1, first=False)

        def lbody(g, _):
            for u in range(2):
                l = 2 * g + u
                wait_gather(l, u)

                @pl.when(l + 2 < L)
                def _():
                    gather(l + 2, u)

                consume(u, first=False)
            return 0

        lax.fori_loop(1, L // 2, lbody, 0)
        pltpu.sync_copy(acc_v, out_hbm.at[pl.ds(base, bpw)])

    return k(ids, table)


def _tc_classifier(sums, cls_w, cls_b, L):
    B, D = sums.shape
    _, OUT = cls_w.shape

    def body(x_ref, w_ref, b_ref, o_ref):
        x = x_ref[...] * (1.0 / L)
        y = jnp.dot(x, w_ref[...], preferred_element_type=jnp.float32)
        o_ref[...] = jax.nn.sigmoid(y + b_ref[...])

    return pl.pallas_call(
        body,
        out_shape=jax.ShapeDtypeStruct((B, OUT), jnp.float32),
    )(sums, cls_w, cls_b.reshape(1, OUT))


def kernel(input_ids, emb_table, cls_w, cls_b):
    ids = input_ids.astype(jnp.int32)
    L, _ = ids.shape
    sums = _sc_gather_sum(ids, emb_table)
    return _tc_classifier(sums, cls_w, cls_b, L)


# speedup vs baseline: 1.4781x; 1.4781x over previous
import functools

import jax
import jax.numpy as jnp
from jax import lax
from jax.experimental import pallas as pl
from jax.experimental.pallas import tpu as pltpu
from jax.experimental.pallas import tpu_sc as plsc

LANES = 16


def _tc_build_pairs(table_t, S, C):
    D, V = table_t.shape
    G = S // C

    def body(a_ref, b_ref, out_ref):
        ya = jnp.swapaxes(a_ref[...], 0, 1)
        yb = jnp.swapaxes(b_ref[...], 0, 1)
        out_ref[...] = jnp.concatenate([ya, yb], axis=1)

    soff = S // C
    gmax = (V + C - 1) // C - 1

    return pl.pallas_call(
        body,
        grid=(G,),
        in_specs=[
            pl.BlockSpec((D, C), lambda g: (0, g)),
            pl.BlockSpec((D, C), lambda g: (0, jnp.minimum(g + soff, gmax))),
        ],
        out_specs=pl.BlockSpec((C, 2 * D), lambda g: (g, 0)),
        out_shape=jax.ShapeDtypeStruct((S, 2 * D), jnp.float32),
    )(table_t, table_t)


def _sc_gather_sum(ids, tablev, S):
    L, B = ids.shape
    _, D2 = tablev.shape
    D = D2 // 2
    NC, NS = 2, 16
    NW = NC * NS
    bpw = B // NW
    CV = D // LANES
    JU = LANES

    mesh = plsc.VectorSubcoreMesh(core_axis_name="c", subcore_axis_name="s")

    @functools.partial(
        pl.kernel,
        out_type=jax.ShapeDtypeStruct((B, D), jnp.float32),
        mesh=mesh,
        scratch_types=[
            pltpu.VMEM((L, bpw), jnp.int32),
            pltpu.VMEM((L, bpw), jnp.int32),
            pltpu.VMEM((2, bpw, D2), jnp.float32),
            pltpu.VMEM((bpw, D), jnp.float32),
            pltpu.SemaphoreType.DMA,
            pltpu.SemaphoreType.DMA,
        ],
        compiler_params=pltpu.CompilerParams(use_tc_tiling_on_sc=True),
    )
    def k(ids_hbm, table_hbm, out_hbm, idx_v, idx2_v, rows_v, acc_v, sem0, sem1):
        wid = lax.axis_index("s") * NC + lax.axis_index("c")
        base = wid * bpw
        pltpu.sync_copy(ids_hbm.at[:, pl.ds(base, bpw)], idx_v)
        sems = (sem0, sem1)

        def hbody(l, _):
            for c in range(bpw // LANES):
                sl = pl.ds(c * LANES, LANES)
                v = idx_v[l, sl]
                idx2_v[l, sl] = jnp.where(v >= S, v - S, v)
            return 0

        lax.fori_loop(0, L, hbody, 0)

        def gather(l, buf):
            pltpu.async_copy(table_hbm.at[idx2_v.at[l]], rows_v.at[buf], sems[buf])

        def wait_gather(l, buf):
            pltpu.make_async_copy(
                table_hbm.at[idx2_v.at[l]], rows_v.at[buf], sems[buf]
            ).wait()

        def consume(l, buf, first):
            def jbody(i, _):
                jb = i * JU
                pv = idx_v[l, pl.ds(jb, JU)]
                for u in range(JU):
                    j = jb + u
                    odd = pv[u] >= S
                    for c in range(CV):
                        sl = pl.ds(c * LANES, LANES)
                        lo = rows_v[buf, j, sl]
                        hi = rows_v[buf, j, pl.ds(D + c * LANES, LANES)]
                        x = jnp.where(odd, hi, lo)
                        if first:
                            acc_v[j, sl] = x
                        else:
                            plsc.addupdate(acc_v.at[j, sl], x)
                return 0

            lax.fori_loop(0, bpw // JU, jbody, 0)

        gather(0, 0)
        gather(1, 1)
        wait_gather(0, 0)
        consume(0, 0, first=True)
        gather(2, 0)
        wait_gather(1, 1)
        consume(1, 1, first=False)
        gather(3, 1)

        def lbody(g, _):
            for u in range(2):
                l = 2 * g + u
                wait_gather(l, u)
                consume(l, u, first=False)

                @pl.when(l + 2 < L)
                def _():
                    gather(l + 2, u)

            return 0

        lax.fori_loop(1, L // 2, lbody, 0)
        pltpu.sync_copy(acc_v, out_hbm.at[pl.ds(base, bpw)])

    return k(ids, tablev)


def _tc_classifier(sums, cls_w, cls_b, L):
    B, D = sums.shape
    _, OUT = cls_w.shape

    def body(x_ref, w_ref, b_ref, o_ref):
        x = x_ref[...] * (1.0 / L)
        y = jnp.dot(x, w_ref[...], preferred_element_type=jnp.float32)
        o_ref[...] = jax.nn.sigmoid(y + b_ref[...])

    return pl.pallas_call(
        body,
        out_shape=jax.ShapeDtypeStruct((B, OUT), jnp.float32),
    )(sums, cls_w, cls_b.reshape(1, OUT))


def kernel(input_ids, emb_table, cls_w, cls_b):
    ids = input_ids.astype(jnp.int32)
    L, _ = ids.shape
    V = emb_table.shape[0]
    C = 2048
    S = C * ((V // 2 + C - 1) // C)
    tablev = _tc_build_pairs(emb_table.T, S, C)
    sums = _sc_gather_sum(ids, tablev, S)
    return _tc_classifier(sums, cls_w, cls_b, L)


# speedup vs baseline: 2.0196x; 1.3663x over previous
import functools

import jax
import jax.numpy as jnp
from jax import lax
from jax.experimental import pallas as pl
from jax.experimental.pallas import tpu as pltpu
from jax.experimental.pallas import tpu_sc as plsc

LANES = 16


def _tc_build_pairs(table_t, S, C):
    D, V = table_t.shape
    G = S // C

    def body(a_ref, b_ref, out_ref):
        ya = jnp.swapaxes(a_ref[...], 0, 1)
        yb = jnp.swapaxes(b_ref[...], 0, 1)
        out_ref[...] = jnp.concatenate([ya, yb], axis=1)

    soff = S // C
    gmax = (V + C - 1) // C - 1

    return pl.pallas_call(
        body,
        grid=(G,),
        in_specs=[
            pl.BlockSpec((D, C), lambda g: (0, g)),
            pl.BlockSpec((D, C), lambda g: (0, jnp.minimum(g + soff, gmax))),
        ],
        out_specs=pl.BlockSpec((C, 2 * D), lambda g: (g, 0)),
        out_shape=jax.ShapeDtypeStruct((S, 2 * D), jnp.float32),
    )(table_t, table_t)


def _sc_gather_sum(ids, table_lin, S):
    L, B = ids.shape
    _, D = table_lin.shape
    NC, NS = 2, 16
    NW = NC * NS
    bpw = B // NW
    CV = D // LANES
    JU = 8

    mesh = plsc.VectorSubcoreMesh(core_axis_name="c", subcore_axis_name="s")

    @functools.partial(
        pl.kernel,
        out_type=jax.ShapeDtypeStruct((B, D), jnp.float32),
        mesh=mesh,
        scratch_types=[
            pltpu.VMEM((L, bpw), jnp.int32),
            pltpu.VMEM((2, bpw, D), jnp.float32),
            pltpu.VMEM((bpw, D), jnp.float32),
            pltpu.SemaphoreType.DMA,
            pltpu.SemaphoreType.DMA,
        ],
        compiler_params=pltpu.CompilerParams(use_tc_tiling_on_sc=False),
    )
    def k(ids_hbm, table_hbm, out_hbm, idx_v, rows_v, acc_v, sem0, sem1):
        wid = lax.axis_index("s") * NC + lax.axis_index("c")
        base = wid * bpw
        pltpu.sync_copy(ids_hbm.at[:, pl.ds(base, bpw)], idx_v)
        sems = (sem0, sem1)

        def hbody(l, _):
            for c in range(bpw // LANES):
                sl = pl.ds(c * LANES, LANES)
                v = idx_v[l, sl]
                idx_v[l, sl] = jnp.where(v >= S, 2 * (v - S) + 1, 2 * v)
            return 0

        lax.fori_loop(0, L, hbody, 0)

        def gather(l, buf):
            pltpu.async_copy(table_hbm.at[idx_v.at[l]], rows_v.at[buf], sems[buf])

        def wait_gather(l, buf):
            pltpu.make_async_copy(
                table_hbm.at[idx_v.at[l]], rows_v.at[buf], sems[buf]
            ).wait()

        def consume(buf, first):
            def jbody(i, _):
                for u in range(JU):
                    j = i * JU + u
                    for c in range(CV):
                        sl = pl.ds(c * LANES, LANES)
                        x = rows_v[buf, j, sl]
                        if first:
                            acc_v[j, sl] = x
                        else:
                            plsc.addupdate(acc_v.at[j, sl], x)
                return 0

            lax.fori_loop(0, bpw // JU, jbody, 0)

        gather(0, 0)
        gather(1, 1)
        wait_gather(0, 0)
        consume(0, first=True)
        gather(2, 0)
        wait_gather(1, 1)
        consume(1, first=False)
        gather(3, 1)

        def lbody(g, _):
            for u in range(2):
                l = 2 * g + u
                wait_gather(l, u)
                consume(u, first=False)

                @pl.when(l + 2 < L)
                def _():
                    gather(l + 2, u)

            return 0

        lax.fori_loop(1, L // 2, lbody, 0)
        pltpu.sync_copy(acc_v, out_hbm.at[pl.ds(base, bpw)])

    return k(ids, table_lin)


def _tc_classifier(sums, cls_w, cls_b, L):
    B, D = sums.shape
    _, OUT = cls_w.shape

    def body(x_ref, w_ref, b_ref, o_ref):
        x = x_ref[...] * (1.0 / L)
        y = jnp.dot(x, w_ref[...], preferred_element_type=jnp.float32)
        o_ref[...] = jax.nn.sigmoid(y + b_ref[...])

    return pl.pallas_call(
        body,
        out_shape=jax.ShapeDtypeStruct((B, OUT), jnp.float32),
    )(sums, cls_w, cls_b.reshape(1, OUT))


def kernel(input_ids, emb_table, cls_w, cls_b):
    ids = input_ids.astype(jnp.int32)
    L, _ = ids.shape
    V, D = emb_table.shape
    C = 4096
    S = C * ((V // 2 + C - 1) // C)
    tablev = _tc_build_pairs(emb_table.T, S, C)
    table_lin = tablev.reshape(2 * S, D)
    sums = _sc_gather_sum(ids, table_lin, S)
    return _tc_classifier(sums, cls_w, cls_b, L)


# speedup vs baseline: 2.0801x; 1.0299x over previous
import functools

import jax
import jax.numpy as jnp
from jax import lax
from jax.experimental import pallas as pl
from jax.experimental.pallas import tpu as pltpu
from jax.experimental.pallas import tpu_sc as plsc

LANES = 16


def _tc_build_packed(table_t, S4, C):
    D, V = table_t.shape
    G = S4 // C
    soff = S4 // C
    gmax = (V + C - 1) // C - 1

    def body(a_ref, b_ref, c_ref, d_ref, out_ref):
        parts = []
        for r in (a_ref, b_ref, c_ref, d_ref):
            x16 = r[...].astype(jnp.bfloat16)
            lo = jax.lax.bitcast_convert_type(x16[: D // 2, :], jnp.uint16)
            hi = jax.lax.bitcast_convert_type(x16[D // 2 :, :], jnp.uint16)
            z = lo.astype(jnp.uint32) | (hi.astype(jnp.uint32) << 16)
            zf = jax.lax.bitcast_convert_type(z, jnp.float32)
            parts.append(jnp.swapaxes(zf, 0, 1))
        out_ref[...] = jnp.concatenate(parts, axis=1)

    def mk_spec(q):
        return pl.BlockSpec(
            (D, C), lambda g: (0, jnp.minimum(g + q * soff, gmax))
        )

    return pl.pallas_call(
        body,
        grid=(G,),
        in_specs=[mk_spec(0), mk_spec(1), mk_spec(2), mk_spec(3)],
        out_specs=pl.BlockSpec((C, 2 * D), lambda g: (g, 0)),
        out_shape=jax.ShapeDtypeStruct((S4, 2 * D), jnp.float32),
    )(table_t, table_t, table_t, table_t)


def _sc_gather_sum(ids, table_lin, S4, D):
    L, B = ids.shape
    W = D // 4
    NC, NS = 2, 16
    NW = NC * NS
    bpw = B // NW
    JU = 8

    mesh = plsc.VectorSubcoreMesh(core_axis_name="c", subcore_axis_name="s")

    @functools.partial(
        pl.kernel,
        out_type=jax.ShapeDtypeStruct((B, D), jnp.float32),
        mesh=mesh,
        scratch_types=[
            pltpu.VMEM((L, bpw), jnp.int32),
            pltpu.VMEM((2, bpw, D // 2), jnp.float32),
            pltpu.VMEM((bpw, D), jnp.float32),
            pltpu.SemaphoreType.DMA,
            pltpu.SemaphoreType.DMA,
        ],
        compiler_params=pltpu.CompilerParams(
            use_tc_tiling_on_sc=False, needs_layout_passes=False
        ),
    )
    def k(ids_hbm, table_hbm, out_hbm, idx_v, rows_v, acc_v, sem0, sem1):
        wid = lax.axis_index("s") * NC + lax.axis_index("c")
        base = wid * bpw
        pltpu.sync_copy(ids_hbm.at[:, pl.ds(base, bpw)], idx_v)
        sems = (sem0, sem1)

        def hbody(l, _):
            for c in range(bpw // LANES):
                sl = pl.ds(c * LANES, LANES)
                v = idx_v[l, sl]
                one = jnp.ones((LANES,), jnp.int32)
                zero = jnp.zeros((LANES,), jnp.int32)
                q = jnp.where(v >= S4, one, zero)
                q = q + jnp.where(v >= 2 * S4, one, zero)
                q = q + jnp.where(v >= 3 * S4, one, zero)
                r = v - q * S4
                idx_v[l, sl] = 4 * r + q
            return 0

        lax.fori_loop(0, L, hbody, 0)

        def gather(l, buf):
            pltpu.async_copy(table_hbm.at[idx_v.at[l]], rows_v.at[buf], sems[buf])

        def wait_gather(l, buf):
            pltpu.make_async_copy(
                table_hbm.at[idx_v.at[l]], rows_v.at[buf], sems[buf]
            ).wait()

        def consume(buf, first):
            def jbody(i, _):
                for u in range(JU):
                    j = i * JU + u
                    w0 = rows_v[buf, j, pl.ds(0, LANES)]
                    w1 = rows_v[buf, j, pl.ds(LANES, LANES)]
                    l0, h0 = plsc.unpack(
                        plsc.bitcast(w0, jnp.bfloat16),
                        format=plsc.PackFormat.INTERLEAVED,
                    )
                    l1, h1 = plsc.unpack(
                        plsc.bitcast(w1, jnp.bfloat16),
                        format=plsc.PackFormat.INTERLEAVED,
                    )
                    for c, part in ((0, l0), (1, l1), (2, h0), (3, h1)):
                        sl = pl.ds(c * LANES, LANES)
                        if first:
                            acc_v[j, sl] = part
                        else:
                            plsc.addupdate(acc_v.at[j, sl], part)
                return 0

            lax.fori_loop(0, bpw // JU, jbody, 0)

        gather(0, 0)
        gather(1, 1)
        wait_gather(0, 0)
        consume(0, first=True)
        gather(2, 0)
        wait_gather(1, 1)
        consume(1, first=False)
        gather(3, 1)

        def lbody(g, _):
            for u in range(2):
                l = 2 * g + u
                wait_gather(l, u)
                consume(u, first=False)

                @pl.when(l + 2 < L)
                def _():
                    gather(l + 2, u)

            return 0

        lax.fori_loop(1, L // 2, lbody, 0)
        pltpu.sync_copy(acc_v, out_hbm.at[pl.ds(base, bpw)])

    return k(ids, table_lin)


def _tc_classifier(sums, cls_w, cls_b, L):
    B, D = sums.shape
    _, OUT = cls_w.shape

    def body(x_ref, w_ref, b_ref, o_ref):
        x = x_ref[...] * (1.0 / L)
        y = jnp.dot(x, w_ref[...], preferred_element_type=jnp.float32)
        o_ref[...] = jax.nn.sigmoid(y + b_ref[...])

    return pl.pallas_call(
        body,
        out_shape=jax.ShapeDtypeStruct((B, OUT), jnp.float32),
    )(sums, cls_w, cls_b.reshape(1, OUT))


def kernel(input_ids, emb_table, cls_w, cls_b):
    ids = input_ids.astype(jnp.int32)
    L, _ = ids.shape
    V, D = emb_table.shape
    C = 4096
    S4 = C * (((V + 3) // 4 + C - 1) // C)
    tablev = _tc_build_packed(emb_table.T, S4, C)
    table_lin = tablev.reshape(4 * S4, D // 2)
    sums = _sc_gather_sum(ids, table_lin, S4, D)
    return _tc_classifier(sums, cls_w, cls_b, L)


# speedup vs baseline: 3.3903x; 1.6299x over previous
import functools

import jax
import jax.numpy as jnp
from jax import lax
from jax.experimental import pallas as pl
from jax.experimental.pallas import tpu as pltpu
from jax.experimental.pallas import tpu_sc as plsc

LANES = 16


def _tc_build_packed(table_t, S4, C):
    D, V = table_t.shape
    G = S4 // C
    soff = S4 // C
    gmax = (V + C - 1) // C - 1

    def body(a_ref, b_ref, c_ref, d_ref, out_ref):
        x = jnp.concatenate(
            [a_ref[...], b_ref[...], c_ref[...], d_ref[...]], axis=0
        )
        x16 = x.astype(jnp.bfloat16)
        w = pltpu.bitcast(x16, jnp.float32)
        out_ref[...] = jnp.swapaxes(w, 0, 1)

    def mk_spec(q):
        return pl.BlockSpec(
            (D, C), lambda g: (0, jnp.minimum(g + q * soff, gmax))
        )

    return pl.pallas_call(
        body,
        grid=(G,),
        in_specs=[mk_spec(0), mk_spec(1), mk_spec(2), mk_spec(3)],
        out_specs=pl.BlockSpec((C, 2 * D), lambda g: (g, 0)),
        out_shape=jax.ShapeDtypeStruct((S4, 2 * D), jnp.float32),
    )(table_t, table_t, table_t, table_t)


def _sc_gather_sum(ids, table_lin, S4, D):
    L, B = ids.shape
    W = D // 4
    NC, NS = 2, 16
    NW = NC * NS
    bpw = B // NW
    JU = 8

    mesh = plsc.VectorSubcoreMesh(core_axis_name="c", subcore_axis_name="s")

    @functools.partial(
        pl.kernel,
        out_type=jax.ShapeDtypeStruct((B, D), jnp.float32),
        mesh=mesh,
        scratch_types=[
            pltpu.VMEM((L, bpw), jnp.int32),
            pltpu.VMEM((2, bpw, D // 2), jnp.float32),
            pltpu.VMEM((bpw, D), jnp.float32),
            pltpu.SemaphoreType.DMA,
            pltpu.SemaphoreType.DMA,
        ],
        compiler_params=pltpu.CompilerParams(
            use_tc_tiling_on_sc=False, needs_layout_passes=False
        ),
    )
    def k(ids_hbm, table_hbm, out_hbm, idx_v, rows_v, acc_v, sem0, sem1):
        wid = lax.axis_index("s") * NC + lax.axis_index("c")
        base = wid * bpw
        pltpu.sync_copy(ids_hbm.at[:, pl.ds(base, bpw)], idx_v)
        sems = (sem0, sem1)

        def hbody(l, _):
            for c in range(bpw // LANES):
                sl = pl.ds(c * LANES, LANES)
                v = idx_v[l, sl]
                one = jnp.ones((LANES,), jnp.int32)
                zero = jnp.zeros((LANES,), jnp.int32)
                q = jnp.where(v >= S4, one, zero)
                q = q + jnp.where(v >= 2 * S4, one, zero)
                q = q + jnp.where(v >= 3 * S4, one, zero)
                r = v - q * S4
                idx_v[l, sl] = 4 * r + q
            return 0

        lax.fori_loop(0, L, hbody, 0)

        def gather(l, buf):
            pltpu.async_copy(table_hbm.at[idx_v.at[l]], rows_v.at[buf], sems[buf])

        def wait_gather(l, buf):
            pltpu.make_async_copy(
                table_hbm.at[idx_v.at[l]], rows_v.at[buf], sems[buf]
            ).wait()

        def consume(buf, first):
            def jbody(i, _):
                for u in range(JU):
                    j = i * JU + u
                    w0 = rows_v[buf, j, pl.ds(0, LANES)]
                    w1 = rows_v[buf, j, pl.ds(LANES, LANES)]
                    l0, h0 = plsc.unpack(
                        plsc.bitcast(w0, jnp.bfloat16),
                        format=plsc.PackFormat.INTERLEAVED,
                    )
                    l1, h1 = plsc.unpack(
                        plsc.bitcast(w1, jnp.bfloat16),
                        format=plsc.PackFormat.INTERLEAVED,
                    )
                    for c, part in ((0, l0), (1, h0), (2, l1), (3, h1)):
                        sl = pl.ds(c * LANES, LANES)
                        if first:
                            acc_v[j, sl] = part
                        else:
                            plsc.addupdate(acc_v.at[j, sl], part)
                return 0

            lax.fori_loop(0, bpw // JU, jbody, 0)

        gather(0, 0)
        gather(1, 1)
        wait_gather(0, 0)
        consume(0, first=True)
        gather(2, 0)
        wait_gather(1, 1)
        consume(1, first=False)
        gather(3, 1)

        def lbody(g, _):
            for u in range(2):
                l = 2 * g + u
                wait_gather(l, u)
                consume(u, first=False)

                @pl.when(l + 2 < L)
                def _():
                    gather(l + 2, u)

            return 0

        lax.fori_loop(1, L // 2, lbody, 0)
        pltpu.sync_copy(acc_v, out_hbm.at[pl.ds(base, bpw)])

    return k(ids, table_lin)


def _tc_classifier(sums, cls_w, cls_b, L):
    B, D = sums.shape
    _, OUT = cls_w.shape

    def body(x_ref, w_ref, b_ref, o_ref):
        x = x_ref[...] * (1.0 / L)
        y = jnp.dot(x, w_ref[...], preferred_element_type=jnp.float32)
        o_ref[...] = jax.nn.sigmoid(y + b_ref[...])

    return pl.pallas_call(
        body,
        out_shape=jax.ShapeDtypeStruct((B, OUT), jnp.float32),
    )(sums, cls_w, cls_b.reshape(1, OUT))


def kernel(input_ids, emb_table, cls_w, cls_b):
    ids = input_ids.astype(jnp.int32)
    L, _ = ids.shape
    V, D = emb_table.shape
    C = 4096
    S4 = C * (((V + 3) // 4 + C - 1) // C)
    tablev = _tc_build_packed(emb_table.T, S4, C)
    table_lin = tablev.reshape(4 * S4, D // 2)
    sums = _sc_gather_sum(ids, table_lin, S4, D)
    perm = []
    for half in range(D // 32):
        perm += [32 * half + 2 * k for k in range(16)]
        perm += [32 * half + 2 * k + 1 for k in range(16)]
    cls_w_p = cls_w[jnp.array(perm, dtype=jnp.int32), :]
    return _tc_classifier(sums, cls_w_p, cls_b, L)
